# Initial kernel scaffold; baseline (speedup 1.0000x reference)
import jax
import jax.numpy as jnp
from jax.experimental import pallas as pl


def kernel(hmap, rreg, iseg, sseg):
    raise NotImplementedError("write your pallas kernel here")


# speedup vs baseline: 9.1183x; 9.1183x over previous
import jax
import jax.numpy as jnp
from jax import lax
from jax.experimental import pallas as pl
from jax.experimental.pallas import tpu as pltpu

_PEAK_T = 0.3
_OFF_T = 5.0 ** 2
_K = 64


def _peaks_kernel(h_ref, r_ref, c0_ref, c1_ref, types_ref, scores_ref,
                  valid_ref):
    h = h_ref[...]
    ninf = jnp.full_like(h, -jnp.inf)
    shl = jnp.concatenate([h[:, 8:], ninf[:, :8]], axis=1)
    shr = jnp.concatenate([ninf[:, :8], h[:, :-8]], axis=1)
    cm = jnp.maximum(h, jnp.maximum(shl, shr))
    up = jnp.concatenate([cm[1:, :], ninf[:1, :]], axis=0)
    dn = jnp.concatenate([ninf[:1, :], cm[:-1, :]], axis=0)
    pooled = jnp.maximum(cm, jnp.maximum(up, dn))
    masked = jnp.where((h == pooled) & (h > _PEAK_T), h, -jnp.inf)

    fidx = (lax.broadcasted_iota(jnp.int32, (64, 512), 0) * 512
            + lax.broadcasted_iota(jnp.int32, (64, 512), 1))

    rr = r_ref[...]
    rrow = lax.broadcasted_iota(jnp.int32, (64, 128), 0)
    rcol = lax.broadcasted_iota(jnp.int32, (64, 128), 1)
    rpix = rrow * 64 + rcol // 2
    rd = rcol % 2

    kiota = lax.broadcasted_iota(jnp.int32, (1, _K), 1)

    def body(k, carry):
        msk, sv, iv, r0v, r1v = carry
        m = jnp.max(msk)
        idx = jnp.min(jnp.where(msk == m, fidx, jnp.int32(2 ** 30)))
        pix = idx // 8
        hit = rpix == pix
        r0 = jnp.sum(jnp.where(hit & (rd == 0), rr, 0.0))
        r1 = jnp.sum(jnp.where(hit & (rd == 1), rr, 0.0))
        sel = kiota == k
        sv = jnp.where(sel, m, sv)
        iv = jnp.where(sel, idx, iv)
        r0v = jnp.where(sel, r0, r0v)
        r1v = jnp.where(sel, r1, r1v)
        msk = jnp.where(fidx == idx, -jnp.inf, msk)
        return msk, sv, iv, r0v, r1v

    init = (masked,
            jnp.full((1, _K), -jnp.inf, jnp.float32),
            jnp.zeros((1, _K), jnp.int32),
            jnp.zeros((1, _K), jnp.float32),
            jnp.zeros((1, _K), jnp.float32))
    _, sv, iv, r0v, r1v = lax.fori_loop(0, _K, body, init)

    pix = iv // 8
    py = (pix // 64).astype(jnp.float32)
    px = (pix % 64).astype(jnp.float32)
    c0_ref[...] = (px + r1v) * 8.0
    c1_ref[...] = (py + r0v) * 8.0
    types_ref[...] = iv % 8
    valid = sv > _PEAK_T
    scores_ref[...] = jnp.where(valid, sv, 0.0)
    valid_ref[...] = valid.astype(jnp.float32)


def _dense_kernel(st_ref, iy_ref, ix_ref, c0_ref, c1_ref, valid_ref,
                  aff_ref, osc_ref, coy_ref, cox_ref):
    rows = iy_ref.shape[0]
    i = pl.program_id(0)
    s0 = st_ref[0]
    m = st_ref[1]
    for c in range(2, 8):
        m = jnp.maximum(m, st_ref[c])
    non_bg = m > s0
    yy = (lax.broadcasted_iota(jnp.int32, (rows, 512), 0)
          + i * rows).astype(jnp.float32)
    xx = lax.broadcasted_iota(jnp.int32, (rows, 512), 1).astype(jnp.float32)
    o0 = iy_ref[...] + yy
    o1 = ix_ref[...] + xx
    mind = jnp.full((rows, 512), jnp.inf, jnp.float32)
    amin = jnp.zeros((rows, 512), jnp.int32)
    for k in range(_K):
        d0 = o0 - c0_ref[0, k]
        d1 = o1 - c1_ref[0, k]
        d = d0 * d0 + d1 * d1
        d = jnp.where(valid_ref[0, k] > 0.5, d, 1e30)
        upd = d < mind
        amin = jnp.where(upd, k, amin)
        mind = jnp.where(upd, d, mind)
    validp = non_bg & (mind < _OFF_T)
    aff_ref[...] = jnp.where(validp, amin, -1)
    osc_ref[...] = jnp.where(validp, mind, 0.0)
    coy_ref[...] = jnp.where(validp, yy, 0.0)
    cox_ref[...] = jnp.where(validp, xx, 0.0)


@jax.jit
def kernel(hmap, rreg, iseg, sseg):
    h2 = hmap[0].reshape(64, 512)
    r2 = rreg[0].reshape(64, 128)
    f32 = jnp.float32
    c0, c1, types, scores, valid = pl.pallas_call(
        _peaks_kernel,
        out_shape=[
            jax.ShapeDtypeStruct((1, _K), f32),
            jax.ShapeDtypeStruct((1, _K), f32),
            jax.ShapeDtypeStruct((1, _K), jnp.int32),
            jax.ShapeDtypeStruct((1, _K), f32),
            jax.ShapeDtypeStruct((1, _K), f32),
        ],
    )(h2, r2)

    sseg_t = jnp.transpose(sseg[0], (2, 0, 1))
    iy = iseg[0, :, :, 1]
    ix = iseg[0, :, :, 0]
    R = 64
    G = 512 // R
    smem = pl.BlockSpec(memory_space=pltpu.SMEM)
    aff, osc, coy, cox = pl.pallas_call(
        _dense_kernel,
        grid=(G,),
        in_specs=[
            pl.BlockSpec((8, R, 512), lambda i: (0, i, 0)),
            pl.BlockSpec((R, 512), lambda i: (i, 0)),
            pl.BlockSpec((R, 512), lambda i: (i, 0)),
            smem, smem, smem,
        ],
        out_specs=[
            pl.BlockSpec((R, 512), lambda i: (i, 0)),
            pl.BlockSpec((R, 512), lambda i: (i, 0)),
            pl.BlockSpec((R, 512), lambda i: (i, 0)),
            pl.BlockSpec((R, 512), lambda i: (i, 0)),
        ],
        out_shape=[
            jax.ShapeDtypeStruct((512, 512), jnp.int32),
            jax.ShapeDtypeStruct((512, 512), f32),
            jax.ShapeDtypeStruct((512, 512), f32),
            jax.ShapeDtypeStruct((512, 512), f32),
        ],
    )(sseg_t, iy, ix, c0, c1, valid)

    centroids = jnp.concatenate([c0.reshape(_K, 1), c1.reshape(_K, 1)],
                                axis=1)
    coords = jnp.stack([coy, cox], axis=-1).reshape(-1, 2)
    return (coords, aff.reshape(-1), centroids, types.reshape(_K),
            scores.reshape(_K), osc.reshape(-1))
